# baseline (speedup 1.0000x reference)
import functools

import jax
import jax.numpy as jnp
from jax import lax
from jax.experimental import pallas as pl
from jax.experimental.pallas import tpu as pltpu
from jax.experimental.pallas import tpu_sc as plsc

N = 10000
E = 320000
D = 128
BN_EPS = 1e-5

NC = 2
NS = 16
NW = NC * NS
E_PER_W = E // NW
K = 40
CHUNKS = E_PER_W // K
G = 50
NG = CHUNKS // G
STRIPE = 624
TAIL = N - NS * STRIPE
LANES = 16


def _prep_body(ex0_ref, ex1_ref, ex2_ref, e0_ref, e1_ref, e2_ref,
               c_ref, comb_ref):
    c_ref[...] = ex0_ref[...] * 64 + ex1_ref[...] * 8 + ex2_ref[...]
    e12 = (e1_ref[...][:, None, :] + e2_ref[...][None, :, :]).reshape(64, D)
    comb_ref[...] = (e0_ref[...][:, None, :] + e12[None, :, :]).reshape(512, D)


_sc_mesh = plsc.VectorSubcoreMesh(core_axis_name="c", subcore_axis_name="s")


@functools.partial(
    pl.kernel,
    out_type=jax.ShapeDtypeStruct((NC, N, D), jnp.float32),
    mesh=_sc_mesh,
    scratch_types=[
        pltpu.VMEM((G, 2, K), jnp.int32),
        pltpu.VMEM((G, K), jnp.int32),
        pltpu.VMEM((K, D), jnp.float32),
        pltpu.VMEM((K, D), jnp.float32),
        pltpu.VMEM((K, D), jnp.float32),
        pltpu.VMEM((K, D), jnp.float32),
        pltpu.VMEM_SHARED((N, D), jnp.float32),
        pltpu.VMEM_SHARED((512, D), jnp.float32),
        pltpu.SemaphoreType.DMA,
        pltpu.SemaphoreType.DMA,
        pltpu.SemaphoreType.DMA,
        pltpu.SemaphoreType.DMA,
        pltpu.SemaphoreType.DMA,
        pltpu.SemaphoreType.DMA,
    ],
)
def _sc_agg(x_hbm, gidx_hbm, didx_hbm, comb_hbm, out_hbm,
            gbulk, dbulk, xb0, xb1, eb0, eb1, acc, combsp,
            gx0, gx1, ge0, ge1, sp0, sp1):
    cid = lax.axis_index("c")
    sid = lax.axis_index("s")
    wid = sid * NC + cid

    xbuf = (xb0, xb1)
    ebuf = (eb0, eb1)
    gxs = (gx0, gx1)
    ges = (ge0, ge1)
    sps = (sp0, sp1)

    @pl.when(sid == 0)
    def _stage_comb():
        pltpu.sync_copy(comb_hbm, combsp)

    def _zrow(r, carry):
        for j in range(D // LANES):
            xb0[r, pl.ds(j * LANES, LANES)] = jnp.zeros((LANES,), jnp.float32)
        return carry
    lax.fori_loop(0, K, _zrow, 0)
    for t in range(STRIPE // K):
        pltpu.sync_copy(xb0, acc.at[pl.ds(sid * STRIPE + t * K, K)])
    rem = STRIPE - (STRIPE // K) * K
    pltpu.sync_copy(xb0.at[pl.ds(0, rem)],
                    acc.at[pl.ds(sid * STRIPE + (STRIPE // K) * K, rem)])

    @pl.when(sid == 0)
    def _zero_tail():
        pltpu.sync_copy(xb0.at[pl.ds(0, TAIL)],
                        acc.at[pl.ds(NS * STRIPE, TAIL)])
    plsc.subcore_barrier()

    def _drain(q):
        pltpu.make_async_copy(xbuf[q], acc.at[dbulk.at[0]], sps[q]).wait()

    def _issue_gathers(i, p):
        pltpu.async_copy(x_hbm.at[gbulk.at[i, 0]], xbuf[p], gxs[p])
        pltpu.async_copy(combsp.at[gbulk.at[i, 1]], ebuf[p], ges[p])

    def _wait_gathers(i, p):
        pltpu.make_async_copy(x_hbm.at[gbulk.at[i, 0]], xbuf[p],
                              gxs[p]).wait()
        pltpu.make_async_copy(combsp.at[gbulk.at[i, 1]], ebuf[p],
                              ges[p]).wait()

    def _compute(p):
        @plsc.parallel_loop(0, K, step=1, unroll=8)
        def _row(k):
            for j in range(D // LANES):
                s = pl.ds(j * LANES, LANES)
                xbuf[p][k, s] = jnp.maximum(xbuf[p][k, s] + ebuf[p][k, s], 0.0)

    def _scatter(i, p):
        pltpu.async_copy(xbuf[p], acc.at[dbulk.at[i]], sps[p], add=True)

    def _group(t, carry):
        @pl.when(t > 0)
        def _drain_group_tail():
            _drain(1)
        pltpu.sync_copy(gidx_hbm.at[wid, pl.ds(t * G, G)], gbulk)
        pltpu.sync_copy(didx_hbm.at[wid, t], dbulk)
        _issue_gathers(0, 0)

        def _pair(u, carry2):
            iA = 2 * u

            @pl.when(u > 0)
            def _drain_a():
                _drain(1)
            _issue_gathers(iA + 1, 1)
            _wait_gathers(iA, 0)
            _compute(0)
            _scatter(iA, 0)

            _drain(0)

            @pl.when(iA + 2 < G)
            def _prefetch_b():
                _issue_gathers(iA + 2, 0)
            _wait_gathers(iA + 1, 1)
            _compute(1)
            _scatter(iA + 1, 1)
            return carry2
        lax.fori_loop(0, G // 2, _pair, 0)
        return carry
    lax.fori_loop(0, NG, _group, 0)

    _drain(1)

    plsc.subcore_barrier()
    pltpu.sync_copy(
        acc.at[pl.ds(sid * STRIPE, STRIPE)],
        out_hbm.at[cid, pl.ds(sid * STRIPE, STRIPE)])

    @pl.when(sid == 0)
    def _dump_tail():
        pltpu.sync_copy(acc.at[pl.ds(NS * STRIPE, TAIL)],
                        out_hbm.at[cid, pl.ds(NS * STRIPE, TAIL)])


RB = 1000
GRID1 = N // RB


def _mlp_body(eps_ref, x_ref, a0_ref, a1_ref, w1_ref, b1_ref,
              g_ref, be_ref, w2_ref, b2_ref, o_ref, h1_s, st_s):
    i = pl.program_id(0)
    r = i % GRID1

    @pl.when(i < GRID1)
    def _phase_a():
        h = (1.0 + eps_ref[0, 0]) * x_ref[...] + a0_ref[...] + a1_ref[...]
        h1 = jnp.dot(h, w1_ref[...], preferred_element_type=jnp.float32)
        h1 = h1 + b1_ref[...]
        h1_s[pl.ds(r * RB, RB), :] = h1

        @pl.when(i == 0)
        def _init():
            st_s[...] = jnp.zeros_like(st_s)

        st_s[0:1, :] += jnp.sum(h1, axis=0, keepdims=True)
        st_s[1:2, :] += jnp.sum(h1 * h1, axis=0, keepdims=True)

    @pl.when(i >= GRID1)
    def _phase_b():
        mu = st_s[0:1, :] * (1.0 / N)
        var = st_s[1:2, :] * (1.0 / N) - mu * mu
        inv = g_ref[...] * lax.rsqrt(var + BN_EPS)
        h1 = h1_s[pl.ds(r * RB, RB), :]
        z = jnp.maximum((h1 - mu) * inv + be_ref[...], 0.0)
        o_ref[...] = (jnp.dot(z, w2_ref[...],
                              preferred_element_type=jnp.float32)
                      + b2_ref[...])


def kernel(x, edge_index, ex, eps_p, W1, b1, gamma, beta, W2, b2,
           emb0, emb1, emb2):
    TD = 2 * D
    EROWS = E // D

    exT = ex.T
    ex0 = exT[0].reshape(EROWS, D)
    ex1 = exT[1].reshape(EROWS, D)
    ex2c = exT[2].reshape(EROWS, D)

    c2d, comb = pl.pallas_call(
        _prep_body,
        out_shape=[
            jax.ShapeDtypeStruct((EROWS, D), jnp.int32),
            jax.ShapeDtypeStruct((512, D), jnp.float32),
        ],
    )(ex0, ex1, ex2c, emb0, emb1, emb2)

    src3 = edge_index[0].reshape(NW, CHUNKS, K)
    dst4 = edge_index[1].reshape(NW, NG, G, K)
    c3 = c2d.reshape(NW, CHUNKS, K)
    gidx = jnp.stack([src3, c3], axis=2)

    agg = _sc_agg(x, gidx, dst4, comb)

    eps_arr = eps_p.reshape(1, 1)
    out = pl.pallas_call(
        _mlp_body,
        grid=(2 * GRID1,),
        in_specs=[
            pl.BlockSpec((1, 1), lambda i: (0, 0)),
            pl.BlockSpec((RB, D), lambda i: (i % GRID1, 0)),
            pl.BlockSpec((RB, D), lambda i: (i % GRID1, 0)),
            pl.BlockSpec((RB, D), lambda i: (i % GRID1, 0)),
            pl.BlockSpec((D, TD), lambda i: (0, 0)),
            pl.BlockSpec((1, TD), lambda i: (0, 0)),
            pl.BlockSpec((1, TD), lambda i: (0, 0)),
            pl.BlockSpec((1, TD), lambda i: (0, 0)),
            pl.BlockSpec((TD, D), lambda i: (0, 0)),
            pl.BlockSpec((1, D), lambda i: (0, 0)),
        ],
        out_specs=pl.BlockSpec((RB, D), lambda i: (i % GRID1, 0)),
        out_shape=jax.ShapeDtypeStruct((N, D), jnp.float32),
        scratch_shapes=[
            pltpu.VMEM((N, TD), jnp.float32),
            pltpu.VMEM((8, TD), jnp.float32),
        ],
    )(eps_arr, x, agg[0], agg[1], W1, b1.reshape(1, TD),
      gamma.reshape(1, TD), beta.reshape(1, TD), W2, b2.reshape(1, D))

    return out


# speedup vs baseline: 1.0891x; 1.0891x over previous
import functools

import jax
import jax.numpy as jnp
from jax import lax
from jax.experimental import pallas as pl
from jax.experimental.pallas import tpu as pltpu
from jax.experimental.pallas import tpu_sc as plsc

N = 10000
E = 320000
D = 128
BN_EPS = 1e-5

NC = 2
NS = 16
NW = NC * NS
E_PER_W = E // NW
K = 40
CHUNKS = E_PER_W // K
G = 50
NG = CHUNKS // G
STRIPE = 624
TAIL = N - NS * STRIPE
LANES = 16


def _prep_body(ex0_ref, ex1_ref, ex2_ref, e0_ref, e1_ref, e2_ref,
               c_ref, comb_ref):
    c_ref[...] = ex0_ref[...] * 64 + ex1_ref[...] * 8 + ex2_ref[...]
    e12 = (e1_ref[...][:, None, :] + e2_ref[...][None, :, :]).reshape(64, D)
    comb_ref[...] = (e0_ref[...][:, None, :] + e12[None, :, :]).reshape(512, D)


_sc_mesh = plsc.VectorSubcoreMesh(core_axis_name="c", subcore_axis_name="s")


@functools.partial(
    pl.kernel,
    out_type=jax.ShapeDtypeStruct((NC, N, D), jnp.float32),
    mesh=_sc_mesh,
    scratch_types=[
        pltpu.VMEM((G, 2, K), jnp.int32),
        pltpu.VMEM((G, K), jnp.int32),
        pltpu.VMEM((K, D), jnp.float32),
        pltpu.VMEM((K, D), jnp.float32),
        pltpu.VMEM((K, D), jnp.float32),
        pltpu.VMEM((K, D), jnp.float32),
        pltpu.VMEM_SHARED((N, D), jnp.float32),
        pltpu.VMEM_SHARED((512, D), jnp.float32),
        pltpu.SemaphoreType.DMA,
        pltpu.SemaphoreType.DMA,
        pltpu.SemaphoreType.DMA,
        pltpu.SemaphoreType.DMA,
        pltpu.SemaphoreType.DMA,
        pltpu.SemaphoreType.DMA,
    ],
)
def _sc_agg(x_hbm, gidx_hbm, didx_hbm, comb_hbm, out_hbm,
            gbulk, dbulk, xb0, xb1, eb0, eb1, acc, combsp,
            gx0, gx1, ge0, ge1, sp0, sp1):
    cid = lax.axis_index("c")
    sid = lax.axis_index("s")
    wid = sid * NC + cid

    xbuf = (xb0, xb1)
    ebuf = (eb0, eb1)
    gxs = (gx0, gx1)
    ges = (ge0, ge1)
    sps = (sp0, sp1)

    @pl.when(sid == 0)
    def _stage_comb():
        pltpu.sync_copy(comb_hbm, combsp)

    def _zrow(r, carry):
        for j in range(D // LANES):
            xb0[r, pl.ds(j * LANES, LANES)] = jnp.zeros((LANES,), jnp.float32)
        return carry
    lax.fori_loop(0, K, _zrow, 0)
    for t in range(STRIPE // K):
        pltpu.sync_copy(xb0, acc.at[pl.ds(sid * STRIPE + t * K, K)])
    rem = STRIPE - (STRIPE // K) * K
    pltpu.sync_copy(xb0.at[pl.ds(0, rem)],
                    acc.at[pl.ds(sid * STRIPE + (STRIPE // K) * K, rem)])

    @pl.when(sid == 0)
    def _zero_tail():
        pltpu.sync_copy(xb0.at[pl.ds(0, TAIL)],
                        acc.at[pl.ds(NS * STRIPE, TAIL)])
    plsc.subcore_barrier()

    def _drain(q):
        pltpu.make_async_copy(xbuf[q], acc.at[dbulk.at[0]], sps[q]).wait()

    def _issue_gathers(i, p):
        pltpu.async_copy(x_hbm.at[gbulk.at[i, 0]], xbuf[p], gxs[p])
        pltpu.async_copy(combsp.at[gbulk.at[i, 1]], ebuf[p], ges[p])

    def _wait_gathers(i, p):
        pltpu.make_async_copy(x_hbm.at[gbulk.at[i, 0]], xbuf[p],
                              gxs[p]).wait()
        pltpu.make_async_copy(combsp.at[gbulk.at[i, 1]], ebuf[p],
                              ges[p]).wait()

    def _compute(p):
        @plsc.parallel_loop(0, K, step=1, unroll=4)
        def _row(k):
            for j in range(D // LANES):
                s = pl.ds(j * LANES, LANES)
                xbuf[p][k, s] = jnp.maximum(xbuf[p][k, s] + ebuf[p][k, s], 0.0)

    def _scatter(i, p):
        pltpu.async_copy(xbuf[p], acc.at[dbulk.at[i]], sps[p], add=True)

    def _group(t, carry):
        @pl.when(t > 0)
        def _drain_group_tail():
            _drain(1)
        pltpu.sync_copy(gidx_hbm.at[wid, pl.ds(t * G, G)], gbulk)
        pltpu.sync_copy(didx_hbm.at[wid, t], dbulk)
        _issue_gathers(0, 0)

        def _pair(u, carry2):
            iA = 2 * u

            @pl.when(u > 0)
            def _drain_a():
                _drain(1)
            _issue_gathers(iA + 1, 1)
            _wait_gathers(iA, 0)
            _compute(0)
            _scatter(iA, 0)

            _drain(0)

            @pl.when(iA + 2 < G)
            def _prefetch_b():
                _issue_gathers(iA + 2, 0)
            _wait_gathers(iA + 1, 1)
            _compute(1)
            _scatter(iA + 1, 1)
            return carry2
        lax.fori_loop(0, G // 2, _pair, 0)
        return carry
    lax.fori_loop(0, NG, _group, 0)

    _drain(1)

    plsc.subcore_barrier()
    pltpu.sync_copy(
        acc.at[pl.ds(sid * STRIPE, STRIPE)],
        out_hbm.at[cid, pl.ds(sid * STRIPE, STRIPE)])

    @pl.when(sid == 0)
    def _dump_tail():
        pltpu.sync_copy(acc.at[pl.ds(NS * STRIPE, TAIL)],
                        out_hbm.at[cid, pl.ds(NS * STRIPE, TAIL)])


RB = 1000
GRID1 = N // RB


def _mlp_body(eps_ref, x_ref, a0_ref, a1_ref, w1_ref, b1_ref,
              g_ref, be_ref, w2_ref, b2_ref, o_ref, h1_s, st_s):
    i = pl.program_id(0)
    r = i % GRID1

    @pl.when(i < GRID1)
    def _phase_a():
        h = (1.0 + eps_ref[0, 0]) * x_ref[...] + a0_ref[...] + a1_ref[...]
        h1 = jnp.dot(h, w1_ref[...], preferred_element_type=jnp.float32)
        h1 = h1 + b1_ref[...]
        h1_s[pl.ds(r * RB, RB), :] = h1

        @pl.when(i == 0)
        def _init():
            st_s[...] = jnp.zeros_like(st_s)

        st_s[0:1, :] += jnp.sum(h1, axis=0, keepdims=True)
        st_s[1:2, :] += jnp.sum(h1 * h1, axis=0, keepdims=True)

    @pl.when(i >= GRID1)
    def _phase_b():
        mu = st_s[0:1, :] * (1.0 / N)
        var = st_s[1:2, :] * (1.0 / N) - mu * mu
        inv = g_ref[...] * lax.rsqrt(var + BN_EPS)
        h1 = h1_s[pl.ds(r * RB, RB), :]
        z = jnp.maximum((h1 - mu) * inv + be_ref[...], 0.0)
        o_ref[...] = (jnp.dot(z, w2_ref[...],
                              preferred_element_type=jnp.float32)
                      + b2_ref[...])


def kernel(x, edge_index, ex, eps_p, W1, b1, gamma, beta, W2, b2,
           emb0, emb1, emb2):
    TD = 2 * D
    EROWS = E // D

    exT = ex.T
    ex0 = exT[0].reshape(EROWS, D)
    ex1 = exT[1].reshape(EROWS, D)
    ex2c = exT[2].reshape(EROWS, D)

    c2d, comb = pl.pallas_call(
        _prep_body,
        out_shape=[
            jax.ShapeDtypeStruct((EROWS, D), jnp.int32),
            jax.ShapeDtypeStruct((512, D), jnp.float32),
        ],
    )(ex0, ex1, ex2c, emb0, emb1, emb2)

    src3 = edge_index[0].reshape(NW, CHUNKS, K)
    dst4 = edge_index[1].reshape(NW, NG, G, K)
    c3 = c2d.reshape(NW, CHUNKS, K)
    gidx = jnp.stack([src3, c3], axis=2)

    agg = _sc_agg(x, gidx, dst4, comb)

    eps_arr = eps_p.reshape(1, 1)
    out = pl.pallas_call(
        _mlp_body,
        grid=(2 * GRID1,),
        in_specs=[
            pl.BlockSpec((1, 1), lambda i: (0, 0)),
            pl.BlockSpec((RB, D), lambda i: (i % GRID1, 0)),
            pl.BlockSpec((RB, D), lambda i: (i % GRID1, 0)),
            pl.BlockSpec((RB, D), lambda i: (i % GRID1, 0)),
            pl.BlockSpec((D, TD), lambda i: (0, 0)),
            pl.BlockSpec((1, TD), lambda i: (0, 0)),
            pl.BlockSpec((1, TD), lambda i: (0, 0)),
            pl.BlockSpec((1, TD), lambda i: (0, 0)),
            pl.BlockSpec((TD, D), lambda i: (0, 0)),
            pl.BlockSpec((1, D), lambda i: (0, 0)),
        ],
        out_specs=pl.BlockSpec((RB, D), lambda i: (i % GRID1, 0)),
        out_shape=jax.ShapeDtypeStruct((N, D), jnp.float32),
        scratch_shapes=[
            pltpu.VMEM((N, TD), jnp.float32),
            pltpu.VMEM((8, TD), jnp.float32),
        ],
    )(eps_arr, x, agg[0], agg[1], W1, b1.reshape(1, TD),
      gamma.reshape(1, TD), beta.reshape(1, TD), W2, b2.reshape(1, D))

    return out
